# baseline (speedup 1.0000x reference)
import functools

import jax
import jax.numpy as jnp
from jax import lax
from jax.experimental import pallas as pl
from jax.experimental.pallas import tpu as pltpu
from jax.experimental.pallas import tpu_sc as plsc

NC = 2
NS = 16
NW = NC * NS
CHUNK = 128


def _mesh():
  return plsc.VectorSubcoreMesh(core_axis_name="c", subcore_axis_name="s")


def _make_deg_kernel(nchunk, nrows, rpt):
  @functools.partial(
      pl.kernel,
      out_type=jax.ShapeDtypeStruct((NC, nrows), jnp.float32),
      mesh=_mesh(),
      compiler_params=pltpu.CompilerParams(use_tc_tiling_on_sc=False,
                                           needs_layout_passes=False),
      scratch_types=[
          pltpu.VMEM((nchunk, CHUNK), jnp.int32),
          pltpu.VMEM((CHUNK,), jnp.float32),
          pltpu.VMEM_SHARED((nrows,), jnp.float32),
      ],
  )
  def deg_kernel(dst_hbm, zrow_hbm, ones_hbm, out_hbm, idx_v, ones_v, deg_sh):
    c = lax.axis_index("c")
    s = lax.axis_index("s")
    w = c * NS + s
    pltpu.sync_copy(dst_hbm.at[pl.ds(w * nchunk, nchunk)], idx_v)
    pltpu.sync_copy(ones_hbm, ones_v)
    pltpu.sync_copy(zrow_hbm, deg_sh.at[pl.ds(s * rpt, rpt)])
    plsc.subcore_barrier()

    def chunk_body(j, carry):
      pltpu.sync_copy(ones_v, deg_sh.at[idx_v.at[j]], add=True)
      return carry

    lax.fori_loop(0, nchunk, chunk_body, 0)
    plsc.subcore_barrier()
    pltpu.sync_copy(deg_sh.at[pl.ds(s * rpt, rpt)],
                    out_hbm.at[c, pl.ds(s * rpt, rpt)])

  return deg_kernel


def _make_agg_kernel_v3(e_pad, lrows, cap, dim_h):
  K = 2048
  nq = e_pad // K
  assert nq % 2 == 0

  @functools.partial(
      pl.kernel,
      out_type=jax.ShapeDtypeStruct((NW, lrows * dim_h), jnp.float32),
      mesh=_mesh(),
      compiler_params=pltpu.CompilerParams(use_tc_tiling_on_sc=False,
                                           needs_layout_passes=False),
      scratch_types=[
          pltpu.VMEM((K,), jnp.int32),
          pltpu.VMEM((K,), jnp.int32),
          pltpu.VMEM((K,), jnp.int32),
          pltpu.VMEM((K,), jnp.int32),
          pltpu.VMEM((cap,), jnp.int32),
          pltpu.VMEM((cap,), jnp.int32),
          pltpu.VMEM((CHUNK, dim_h), jnp.float32),
          pltpu.VMEM((CHUNK, dim_h), jnp.float32),
          pltpu.VMEM((lrows * dim_h,), jnp.float32),
          pltpu.SemaphoreType.DMA,
          pltpu.SemaphoreType.DMA,
          pltpu.SemaphoreType.DMA,
          pltpu.SemaphoreType.DMA,
          pltpu.SemaphoreType.DMA,
          pltpu.SemaphoreType.DMA,
      ],
  )
  def agg_kernel(hp_hbm, src_hbm, dst_hbm, zacc_hbm, fill_src_hbm,
                 fill_loc_hbm, out_hbm,
                 sbuf0, dbuf0, sbuf1, dbuf1, fsrc, floc, buf0, buf1, acc,
                 qsem0, qsem1, qsem2, qsem3, gsem0, gsem1):
    c = lax.axis_index("c")
    s = lax.axis_index("s")
    w = c * NS + s
    sbufs = ((sbuf0, dbuf0), (sbuf1, dbuf1))
    qsems = ((qsem0, qsem1), (qsem2, qsem3))
    bufs = (buf0, buf1)
    gsems = (gsem0, gsem1)

    pltpu.sync_copy(zacc_hbm, acc)
    pltpu.sync_copy(fill_src_hbm, fsrc)
    pltpu.sync_copy(fill_loc_hbm, floc)

    def issue_scan(q, p):
      pltpu.async_copy(src_hbm.at[pl.ds(q * K, K)], sbufs[p][0], qsems[p][0])
      pltpu.async_copy(dst_hbm.at[pl.ds(q * K, K)], sbufs[p][1], qsems[p][1])

    def wait_scan(q, p):
      pltpu.make_async_copy(src_hbm.at[pl.ds(q * K, K)], sbufs[p][0],
                            qsems[p][0]).wait()
      pltpu.make_async_copy(dst_hbm.at[pl.ds(q * K, K)], sbufs[p][1],
                            qsems[p][1]).wait()

    issue_scan(0, 0)

    def scan_pair(i, cnt):
      for p in range(2):
        q = 2 * i + p
        wait_scan(q, p)

        @pl.when(q + 1 < nq)
        def _():
          issue_scan(q + 1, 1 - p)

        sv, dv = sbufs[p]

        def vreg_step(v, cnt2):
          for u in range(4):
            off = v * 64 + u * 16
            d = dv[pl.ds(off, 16)]
            srcv = sv[pl.ds(off, 16)]
            m = (d & (NW - 1)) == w
            pc = plsc.all_reduce_population_count(m)
            plsc.store_compressed(fsrc.at[pl.ds(cnt2, 16)], srcv, mask=m)
            plsc.store_compressed(floc.at[pl.ds(cnt2, 16)], d >> 5, mask=m)
            cnt2 = cnt2 + pc[0]
          return cnt2

        cnt = lax.fori_loop(0, K // 64, vreg_step, cnt)
      return cnt

    lax.fori_loop(0, nq // 2, scan_pair, 0)

    nch = cap // CHUNK

    def issue_gather(k, b):
      pltpu.async_copy(hp_hbm.at[fsrc.at[pl.ds(k * CHUNK, CHUNK)]],
                       bufs[b], gsems[b])

    def wait_gather(k, b):
      pltpu.make_async_copy(hp_hbm.at[fsrc.at[pl.ds(k * CHUNK, CHUNK)]],
                            bufs[b], gsems[b]).wait()

    def accum(k, b):
      buf = bufs[b]
      for g in range(CHUNK // 16):
        lvec = floc[pl.ds(k * CHUNK + g * 16, 16)]
        base = lvec * dim_h
        eid = lax.iota(jnp.int32, 16) + g * 16

        def col(cc, carry):
          ccv = jnp.full((16,), cc, jnp.int32)
          vals = plsc.load_gather(buf, [eid, ccv])
          plsc.addupdate_scatter(acc, [base + cc], vals)
          return carry

        lax.fori_loop(0, dim_h, col, 0)

    issue_gather(0, 0)

    def gather_pair(i, carry):
      for b in range(2):
        k = 2 * i + b
        wait_gather(k, b)

        @pl.when(k + 1 < nch)
        def _():
          issue_gather(k + 1, 1 - b)

        accum(k, b)
      return carry

    lax.fori_loop(0, nch // 2, gather_pair, 0)

    pltpu.sync_copy(acc, out_hbm.at[w])

  return agg_kernel


def _hp_body(x_ref, w1_ref, dega_ref, degb_ref, o_ref):
  deg = dega_ref[...] + degb_ref[...] + 1.0
  d = lax.rsqrt(deg)
  o_ref[...] = jnp.dot(x_ref[...], w1_ref[...],
                       preferred_element_type=jnp.float32) * d


def _head_body(sa_ref, hp_ref, dega_ref, degb_ref, bf_ref, b1_ref,
               w2_ref, b2_ref, o_ref, sums, counts):
  j = pl.program_id(0)

  @pl.when(j == 0)
  def _init():
    sums[...] = jnp.zeros_like(sums)
    counts[...] = jnp.zeros_like(counts)

  d = lax.rsqrt(dega_ref[...] + degb_ref[...] + 1.0)
  h2 = d * (sa_ref[...] + hp_ref[...]) + b1_ref[...]
  h2 = jnp.maximum(h2, 0.0)
  gids = lax.broadcasted_iota(jnp.int32, (1, sums.shape[0]), 1
                              ).astype(jnp.float32)
  onehot = (bf_ref[...] == gids).astype(jnp.float32)
  sums[...] += jnp.dot(onehot.T, h2, preferred_element_type=jnp.float32)
  counts[...] += jnp.sum(onehot, axis=0, keepdims=True)

  @pl.when(j == pl.num_programs(0) - 1)
  def _finish():
    hg = sums[...] / jnp.maximum(counts[...], 1.0).T
    logits = jnp.dot(hg, w2_ref[...],
                     preferred_element_type=jnp.float32) + b2_ref[...]
    m = jnp.max(logits, axis=1, keepdims=True)
    lse = jnp.log(jnp.sum(jnp.exp(logits - m), axis=1, keepdims=True)) + m
    o_ref[...] = logits - lse


def kernel(x, edge_index, batch, W1, b1, W2, b2):
  n, d_feat = x.shape
  dim_h = W1.shape[1]
  n_classes = W2.shape[1]
  e = edge_index.shape[1]
  n_graphs = 128

  nchunk = -(-e // (NW * CHUNK))
  nchunk = (nchunk + 7) // 8 * 8
  e_pad = NW * CHUNK * nchunk
  src = edge_index[0].astype(jnp.int32)
  dst = edge_index[1].astype(jnp.int32)
  pad = e_pad - e
  dst2d = jnp.concatenate([dst, jnp.full((pad,), n, jnp.int32)]
                          ).reshape(NW * nchunk, CHUNK)

  rpt = ((-(-(n + 1) // NS)) + 7) // 8 * 8
  nrows = rpt * NS

  zrow = jnp.zeros((rpt,), jnp.float32)
  ones = jnp.ones((CHUNK,), jnp.float32)
  deg_parts = _make_deg_kernel(nchunk, nrows, rpt)(dst2d, zrow, ones)
  dega = deg_parts[0, :n].reshape(n, 1)
  degb = deg_parts[1, :n].reshape(n, 1)

  nb = 10
  blk = n // nb
  hp = pl.pallas_call(
      _hp_body,
      grid=(nb,),
      in_specs=[
          pl.BlockSpec((blk, d_feat), lambda i: (i, 0)),
          pl.BlockSpec((d_feat, dim_h), lambda i: (0, 0)),
          pl.BlockSpec((blk, 1), lambda i: (i, 0)),
          pl.BlockSpec((blk, 1), lambda i: (i, 0)),
      ],
      out_specs=pl.BlockSpec((blk, dim_h), lambda i: (i, 0)),
      out_shape=jax.ShapeDtypeStruct((n, dim_h), jnp.float32),
  )(x, W1, dega, degb)

  e_pad4 = -(-e // 4096) * 4096
  padn = e_pad4 - e
  srcf = jnp.concatenate([src, jnp.zeros((padn,), jnp.int32)])
  dstf = jnp.concatenate([dst, jnp.full((padn,), n, jnp.int32)])
  lrows = -(-(n + 1) // NW) + 1
  cap = 12288
  zacc = jnp.zeros((lrows * dim_h,), jnp.float32)
  fill_src = jnp.zeros((cap,), jnp.int32)
  fill_loc = jnp.full((cap,), lrows - 1, jnp.int32)
  aggf = _make_agg_kernel_v3(e_pad4, lrows, cap, dim_h)(
      hp, srcf, dstf, zacc, fill_src, fill_loc)
  sa = aggf.reshape(NW, lrows, dim_h).transpose(1, 0, 2).reshape(
      NW * lrows, dim_h)[:n]

  bf = batch.astype(jnp.float32).reshape(n, 1)
  out = pl.pallas_call(
      _head_body,
      grid=(nb,),
      in_specs=[
          pl.BlockSpec((blk, dim_h), lambda i: (i, 0)),
          pl.BlockSpec((blk, dim_h), lambda i: (i, 0)),
          pl.BlockSpec((blk, 1), lambda i: (i, 0)),
          pl.BlockSpec((blk, 1), lambda i: (i, 0)),
          pl.BlockSpec((blk, 1), lambda i: (i, 0)),
          pl.BlockSpec((1, dim_h), lambda i: (0, 0)),
          pl.BlockSpec((dim_h, n_classes), lambda i: (0, 0)),
          pl.BlockSpec((1, n_classes), lambda i: (0, 0)),
      ],
      out_specs=pl.BlockSpec((n_graphs, n_classes), lambda i: (0, 0)),
      out_shape=jax.ShapeDtypeStruct((n_graphs, n_classes), jnp.float32),
      scratch_shapes=[
          pltpu.VMEM((n_graphs, dim_h), jnp.float32),
          pltpu.VMEM((1, n_graphs), jnp.float32),
      ],
  )(sa, hp, dega, degb, bf, b1.reshape(1, dim_h), W2,
    b2.reshape(1, n_classes))
  return out


# speedup vs baseline: 1.0912x; 1.0912x over previous
import functools

import jax
import jax.numpy as jnp
from jax import lax
from jax.experimental import pallas as pl
from jax.experimental.pallas import tpu as pltpu
from jax.experimental.pallas import tpu_sc as plsc

NC = 2
NS = 16
NW = NC * NS
CHUNK = 128


def _mesh():
  return plsc.VectorSubcoreMesh(core_axis_name="c", subcore_axis_name="s")


def _make_deg_kernel(nchunk, nrows, rpt):
  @functools.partial(
      pl.kernel,
      out_type=jax.ShapeDtypeStruct((NC, nrows), jnp.float32),
      mesh=_mesh(),
      compiler_params=pltpu.CompilerParams(use_tc_tiling_on_sc=False,
                                           needs_layout_passes=False),
      scratch_types=[
          pltpu.VMEM((nchunk, CHUNK), jnp.int32),
          pltpu.VMEM((CHUNK,), jnp.float32),
          pltpu.VMEM_SHARED((nrows,), jnp.float32),
      ],
  )
  def deg_kernel(dst_hbm, zrow_hbm, ones_hbm, out_hbm, idx_v, ones_v, deg_sh):
    c = lax.axis_index("c")
    s = lax.axis_index("s")
    w = c * NS + s
    pltpu.sync_copy(dst_hbm.at[pl.ds(w * nchunk, nchunk)], idx_v)
    pltpu.sync_copy(ones_hbm, ones_v)
    pltpu.sync_copy(zrow_hbm, deg_sh.at[pl.ds(s * rpt, rpt)])
    plsc.subcore_barrier()

    def chunk_body(j, carry):
      pltpu.sync_copy(ones_v, deg_sh.at[idx_v.at[j]], add=True)
      return carry

    lax.fori_loop(0, nchunk, chunk_body, 0)
    plsc.subcore_barrier()
    pltpu.sync_copy(deg_sh.at[pl.ds(s * rpt, rpt)],
                    out_hbm.at[c, pl.ds(s * rpt, rpt)])

  return deg_kernel


def _make_agg_kernel_v3(e_pad, lrows, cap, dim_h):
  K = 2048
  nq = e_pad // K
  assert nq % 2 == 0

  @functools.partial(
      pl.kernel,
      out_type=jax.ShapeDtypeStruct((NW, lrows * dim_h), jnp.float32),
      mesh=_mesh(),
      compiler_params=pltpu.CompilerParams(use_tc_tiling_on_sc=False,
                                           needs_layout_passes=False),
      scratch_types=[
          pltpu.VMEM((K,), jnp.int32),
          pltpu.VMEM((K,), jnp.int32),
          pltpu.VMEM((K,), jnp.int32),
          pltpu.VMEM((K,), jnp.int32),
          pltpu.VMEM((cap,), jnp.int32),
          pltpu.VMEM((cap,), jnp.int32),
          pltpu.VMEM((CHUNK, dim_h), jnp.float32),
          pltpu.VMEM((CHUNK, dim_h), jnp.float32),
          pltpu.VMEM((lrows * dim_h,), jnp.float32),
          pltpu.SemaphoreType.DMA,
          pltpu.SemaphoreType.DMA,
          pltpu.SemaphoreType.DMA,
          pltpu.SemaphoreType.DMA,
          pltpu.SemaphoreType.DMA,
          pltpu.SemaphoreType.DMA,
      ],
  )
  def agg_kernel(hp_hbm, src_hbm, dst_hbm, zacc_hbm, fill_src_hbm,
                 fill_loc_hbm, out_hbm,
                 sbuf0, dbuf0, sbuf1, dbuf1, fsrc, floc, buf0, buf1, acc,
                 qsem0, qsem1, qsem2, qsem3, gsem0, gsem1):
    c = lax.axis_index("c")
    s = lax.axis_index("s")
    w = c * NS + s
    sbufs = ((sbuf0, dbuf0), (sbuf1, dbuf1))
    qsems = ((qsem0, qsem1), (qsem2, qsem3))
    bufs = (buf0, buf1)
    gsems = (gsem0, gsem1)

    pltpu.sync_copy(zacc_hbm, acc)
    pltpu.sync_copy(fill_src_hbm, fsrc)
    pltpu.sync_copy(fill_loc_hbm, floc)

    def issue_scan(q, p):
      pltpu.async_copy(src_hbm.at[pl.ds(q * K, K)], sbufs[p][0], qsems[p][0])
      pltpu.async_copy(dst_hbm.at[pl.ds(q * K, K)], sbufs[p][1], qsems[p][1])

    def wait_scan(q, p):
      pltpu.make_async_copy(src_hbm.at[pl.ds(q * K, K)], sbufs[p][0],
                            qsems[p][0]).wait()
      pltpu.make_async_copy(dst_hbm.at[pl.ds(q * K, K)], sbufs[p][1],
                            qsems[p][1]).wait()

    issue_scan(0, 0)

    def scan_pair(i, cnt):
      for p in range(2):
        q = 2 * i + p
        wait_scan(q, p)

        @pl.when(q + 1 < nq)
        def _():
          issue_scan(q + 1, 1 - p)

        sv, dv = sbufs[p]

        def vreg_step(v, cnt2):
          for u in range(4):
            off = v * 64 + u * 16
            d = dv[pl.ds(off, 16)]
            srcv = sv[pl.ds(off, 16)]
            m = (d & (NW - 1)) == w
            pc = plsc.all_reduce_population_count(m)
            plsc.store_compressed(fsrc.at[pl.ds(cnt2, 16)], srcv, mask=m)
            plsc.store_compressed(floc.at[pl.ds(cnt2, 16)], d >> 5, mask=m)
            cnt2 = cnt2 + pc[0]
          return cnt2

        cnt = lax.fori_loop(0, K // 64, vreg_step, cnt)
      return cnt

    lax.fori_loop(0, nq // 2, scan_pair, 0)

    nch = cap // CHUNK

    def issue_gather(k, b):
      pltpu.async_copy(hp_hbm.at[fsrc.at[pl.ds(k * CHUNK, CHUNK)]],
                       bufs[b], gsems[b])

    def wait_gather(k, b):
      pltpu.make_async_copy(hp_hbm.at[fsrc.at[pl.ds(k * CHUNK, CHUNK)]],
                            bufs[b], gsems[b]).wait()

    def accum(k, b):
      buf = bufs[b]
      for g in range(CHUNK // 16):
        lvec = floc[pl.ds(k * CHUNK + g * 16, 16)]
        base = lvec * dim_h
        eid = lax.iota(jnp.int32, 16) + g * 16

        @plsc.parallel_loop(0, dim_h, 8, unroll=4)
        def _cols(cc):
          for u in range(8):
            ccv = jnp.full((16,), cc + u, jnp.int32)
            vals = plsc.load_gather(buf, [eid, ccv])
            plsc.addupdate_scatter(acc, [base + cc + u], vals)

    issue_gather(0, 0)

    def gather_pair(i, carry):
      for b in range(2):
        k = 2 * i + b
        wait_gather(k, b)

        @pl.when(k + 1 < nch)
        def _():
          issue_gather(k + 1, 1 - b)

        accum(k, b)
      return carry

    lax.fori_loop(0, nch // 2, gather_pair, 0)

    pltpu.sync_copy(acc, out_hbm.at[w])

  return agg_kernel


def _hp_body(x_ref, w1_ref, dega_ref, degb_ref, o_ref):
  deg = dega_ref[...] + degb_ref[...] + 1.0
  d = lax.rsqrt(deg)
  o_ref[...] = jnp.dot(x_ref[...], w1_ref[...],
                       preferred_element_type=jnp.float32) * d


def _head_body(sa_ref, hp_ref, dega_ref, degb_ref, bf_ref, b1_ref,
               w2_ref, b2_ref, o_ref, sums, counts):
  j = pl.program_id(0)

  @pl.when(j == 0)
  def _init():
    sums[...] = jnp.zeros_like(sums)
    counts[...] = jnp.zeros_like(counts)

  d = lax.rsqrt(dega_ref[...] + degb_ref[...] + 1.0)
  h2 = d * (sa_ref[...] + hp_ref[...]) + b1_ref[...]
  h2 = jnp.maximum(h2, 0.0)
  gids = lax.broadcasted_iota(jnp.int32, (1, sums.shape[0]), 1
                              ).astype(jnp.float32)
  onehot = (bf_ref[...] == gids).astype(jnp.float32)
  sums[...] += jnp.dot(onehot.T, h2, preferred_element_type=jnp.float32)
  counts[...] += jnp.sum(onehot, axis=0, keepdims=True)

  @pl.when(j == pl.num_programs(0) - 1)
  def _finish():
    hg = sums[...] / jnp.maximum(counts[...], 1.0).T
    logits = jnp.dot(hg, w2_ref[...],
                     preferred_element_type=jnp.float32) + b2_ref[...]
    m = jnp.max(logits, axis=1, keepdims=True)
    lse = jnp.log(jnp.sum(jnp.exp(logits - m), axis=1, keepdims=True)) + m
    o_ref[...] = logits - lse


def kernel(x, edge_index, batch, W1, b1, W2, b2):
  n, d_feat = x.shape
  dim_h = W1.shape[1]
  n_classes = W2.shape[1]
  e = edge_index.shape[1]
  n_graphs = 128

  nchunk = -(-e // (NW * CHUNK))
  nchunk = (nchunk + 7) // 8 * 8
  e_pad = NW * CHUNK * nchunk
  src = edge_index[0].astype(jnp.int32)
  dst = edge_index[1].astype(jnp.int32)
  pad = e_pad - e
  dst2d = jnp.concatenate([dst, jnp.full((pad,), n, jnp.int32)]
                          ).reshape(NW * nchunk, CHUNK)

  rpt = ((-(-(n + 1) // NS)) + 7) // 8 * 8
  nrows = rpt * NS

  zrow = jnp.zeros((rpt,), jnp.float32)
  ones = jnp.ones((CHUNK,), jnp.float32)
  deg_parts = _make_deg_kernel(nchunk, nrows, rpt)(dst2d, zrow, ones)
  dega = deg_parts[0, :n].reshape(n, 1)
  degb = deg_parts[1, :n].reshape(n, 1)

  nb = 10
  blk = n // nb
  hp = pl.pallas_call(
      _hp_body,
      grid=(nb,),
      in_specs=[
          pl.BlockSpec((blk, d_feat), lambda i: (i, 0)),
          pl.BlockSpec((d_feat, dim_h), lambda i: (0, 0)),
          pl.BlockSpec((blk, 1), lambda i: (i, 0)),
          pl.BlockSpec((blk, 1), lambda i: (i, 0)),
      ],
      out_specs=pl.BlockSpec((blk, dim_h), lambda i: (i, 0)),
      out_shape=jax.ShapeDtypeStruct((n, dim_h), jnp.float32),
  )(x, W1, dega, degb)

  e_pad4 = -(-e // 4096) * 4096
  padn = e_pad4 - e
  srcf = jnp.concatenate([src, jnp.zeros((padn,), jnp.int32)])
  dstf = jnp.concatenate([dst, jnp.full((padn,), n, jnp.int32)])
  lrows = -(-(n + 1) // NW) + 1
  cap = 12288
  zacc = jnp.zeros((lrows * dim_h,), jnp.float32)
  fill_src = jnp.zeros((cap,), jnp.int32)
  fill_loc = jnp.full((cap,), lrows - 1, jnp.int32)
  aggf = _make_agg_kernel_v3(e_pad4, lrows, cap, dim_h)(
      hp, srcf, dstf, zacc, fill_src, fill_loc)
  sa = aggf.reshape(NW, lrows, dim_h).transpose(1, 0, 2).reshape(
      NW * lrows, dim_h)[:n]

  bf = batch.astype(jnp.float32).reshape(n, 1)
  out = pl.pallas_call(
      _head_body,
      grid=(nb,),
      in_specs=[
          pl.BlockSpec((blk, dim_h), lambda i: (i, 0)),
          pl.BlockSpec((blk, dim_h), lambda i: (i, 0)),
          pl.BlockSpec((blk, 1), lambda i: (i, 0)),
          pl.BlockSpec((blk, 1), lambda i: (i, 0)),
          pl.BlockSpec((blk, 1), lambda i: (i, 0)),
          pl.BlockSpec((1, dim_h), lambda i: (0, 0)),
          pl.BlockSpec((dim_h, n_classes), lambda i: (0, 0)),
          pl.BlockSpec((1, n_classes), lambda i: (0, 0)),
      ],
      out_specs=pl.BlockSpec((n_graphs, n_classes), lambda i: (0, 0)),
      out_shape=jax.ShapeDtypeStruct((n_graphs, n_classes), jnp.float32),
      scratch_shapes=[
          pltpu.VMEM((n_graphs, dim_h), jnp.float32),
          pltpu.VMEM((1, n_graphs), jnp.float32),
      ],
  )(sa, hp, dega, degb, bf, b1.reshape(1, dim_h), W2,
    b2.reshape(1, n_classes))
  return out


# speedup vs baseline: 7.0774x; 6.4860x over previous
import functools

import jax
import jax.numpy as jnp
from jax import lax
from jax.experimental import pallas as pl
from jax.experimental.pallas import tpu as pltpu
from jax.experimental.pallas import tpu_sc as plsc

NC = 2
NS = 16
NW = NC * NS
CHUNK = 128


def _mesh():
  return plsc.VectorSubcoreMesh(core_axis_name="c", subcore_axis_name="s")


def _make_deg_kernel(nchunk, nrows, rpt):
  @functools.partial(
      pl.kernel,
      out_type=jax.ShapeDtypeStruct((NC, nrows), jnp.float32),
      mesh=_mesh(),
      compiler_params=pltpu.CompilerParams(use_tc_tiling_on_sc=False),
      scratch_types=[
          pltpu.VMEM((nchunk, CHUNK), jnp.int32),
          pltpu.VMEM((CHUNK,), jnp.float32),
          pltpu.VMEM_SHARED((nrows,), jnp.float32),
      ],
  )
  def deg_kernel(dst_hbm, zrow_hbm, ones_hbm, out_hbm, idx_v, ones_v, deg_sh):
    c = lax.axis_index("c")
    s = lax.axis_index("s")
    w = c * NS + s
    pltpu.sync_copy(dst_hbm.at[pl.ds(w * nchunk, nchunk)], idx_v)
    pltpu.sync_copy(ones_hbm, ones_v)
    pltpu.sync_copy(zrow_hbm, deg_sh.at[pl.ds(s * rpt, rpt)])
    plsc.subcore_barrier()

    def chunk_body(j, carry):
      pltpu.sync_copy(ones_v, deg_sh.at[idx_v.at[j]], add=True)
      return carry

    lax.fori_loop(0, nchunk, chunk_body, 0)
    plsc.subcore_barrier()
    pltpu.sync_copy(deg_sh.at[pl.ds(s * rpt, rpt)],
                    out_hbm.at[c, pl.ds(s * rpt, rpt)])

  return deg_kernel


def _make_agg_kernel(nchunk, nrows, rpt, dim_h):
  nhalf = nchunk // 2

  @functools.partial(
      pl.kernel,
      out_type=jax.ShapeDtypeStruct((NC, nrows, dim_h), jnp.float32),
      mesh=_mesh(),
      scratch_types=[
          pltpu.VMEM((nhalf, CHUNK), jnp.int32),
          pltpu.VMEM((nhalf, CHUNK), jnp.int32),
          pltpu.VMEM((CHUNK, dim_h), jnp.float32),
          pltpu.VMEM((CHUNK, dim_h), jnp.float32),
          pltpu.VMEM_SHARED((nrows, dim_h), jnp.float32),
          pltpu.SemaphoreType.DMA,
          pltpu.SemaphoreType.DMA,
      ],
  )
  def agg_kernel(hp_hbm, src_hbm, dst_hbm, zrows_hbm, out_hbm,
                 sidx_v, didx_v, buf0, buf1, agg_sh, gsem0, gsem1):
    c = lax.axis_index("c")
    s = lax.axis_index("s")
    w = c * NS + s
    pltpu.sync_copy(zrows_hbm, agg_sh.at[pl.ds(s * rpt, rpt)])
    plsc.subcore_barrier()

    bufs = (buf0, buf1)
    gsems = (gsem0, gsem1)

    def wait_gather(b, j):
      pltpu.make_async_copy(hp_hbm.at[sidx_v.at[j]], bufs[b], gsems[b]).wait()

    for h in range(2):
      base = w * nchunk + h * nhalf
      pltpu.sync_copy(src_hbm.at[pl.ds(base, nhalf)], sidx_v)
      pltpu.sync_copy(dst_hbm.at[pl.ds(base, nhalf)], didx_v)
      pltpu.async_copy(hp_hbm.at[sidx_v.at[0]], buf0, gsem0)

      def pair_body(i, carry):
        for b in range(2):
          j = 2 * i + b
          nb = 1 - b
          wait_gather(b, j)

          @pl.when(j + 1 < nhalf)
          def _():
            pltpu.async_copy(hp_hbm.at[sidx_v.at[j + 1]], bufs[nb], gsems[nb])

          pltpu.sync_copy(bufs[b], agg_sh.at[didx_v.at[j]], add=True)
        return carry

      lax.fori_loop(0, nhalf // 2, pair_body, 0)

    plsc.subcore_barrier()
    pltpu.sync_copy(agg_sh.at[pl.ds(s * rpt, rpt)],
                    out_hbm.at[c, pl.ds(s * rpt, rpt)])

  return agg_kernel


def _hp_body(x_ref, w1_ref, dega_ref, degb_ref, o_ref):
  deg = dega_ref[...] + degb_ref[...] + 1.0
  d = lax.rsqrt(deg)
  o_ref[...] = jnp.dot(x_ref[...], w1_ref[...],
                       preferred_element_type=jnp.float32) * d


def _head_body(sa_ref, sb_ref, hp_ref, dega_ref, degb_ref, bf_ref, b1_ref,
               w2_ref, b2_ref, o_ref, sums, counts):
  j = pl.program_id(0)

  @pl.when(j == 0)
  def _init():
    sums[...] = jnp.zeros_like(sums)
    counts[...] = jnp.zeros_like(counts)

  d = lax.rsqrt(dega_ref[...] + degb_ref[...] + 1.0)
  h2 = d * (sa_ref[...] + sb_ref[...] + hp_ref[...]) + b1_ref[...]
  h2 = jnp.maximum(h2, 0.0)
  gids = lax.broadcasted_iota(jnp.int32, (1, sums.shape[0]), 1
                              ).astype(jnp.float32)
  onehot = (bf_ref[...] == gids).astype(jnp.float32)
  sums[...] += jnp.dot(onehot.T, h2, preferred_element_type=jnp.float32)
  counts[...] += jnp.sum(onehot, axis=0, keepdims=True)

  @pl.when(j == pl.num_programs(0) - 1)
  def _finish():
    hg = sums[...] / jnp.maximum(counts[...], 1.0).T
    logits = jnp.dot(hg, w2_ref[...],
                     preferred_element_type=jnp.float32) + b2_ref[...]
    m = jnp.max(logits, axis=1, keepdims=True)
    lse = jnp.log(jnp.sum(jnp.exp(logits - m), axis=1, keepdims=True)) + m
    o_ref[...] = logits - lse


def kernel(x, edge_index, batch, W1, b1, W2, b2):
  n, d_feat = x.shape
  dim_h = W1.shape[1]
  n_classes = W2.shape[1]
  e = edge_index.shape[1]
  n_graphs = 128

  nchunk = -(-e // (NW * CHUNK))
  nchunk = (nchunk + 7) // 8 * 8
  e_pad = NW * CHUNK * nchunk
  src = edge_index[0].astype(jnp.int32)
  dst = edge_index[1].astype(jnp.int32)
  pad = e_pad - e
  src2d = jnp.concatenate([src, jnp.zeros((pad,), jnp.int32)]
                          ).reshape(NW * nchunk, CHUNK)
  dst2d = jnp.concatenate([dst, jnp.full((pad,), n, jnp.int32)]
                          ).reshape(NW * nchunk, CHUNK)

  rpt = ((-(-(n + 1) // NS)) + 7) // 8 * 8
  nrows = rpt * NS

  zrow = jnp.zeros((rpt,), jnp.float32)
  ones = jnp.ones((CHUNK,), jnp.float32)
  deg_parts = _make_deg_kernel(nchunk, nrows, rpt)(dst2d, zrow, ones)
  dega = deg_parts[0, :n].reshape(n, 1)
  degb = deg_parts[1, :n].reshape(n, 1)

  nb = 10
  blk = n // nb
  hp = pl.pallas_call(
      _hp_body,
      grid=(nb,),
      in_specs=[
          pl.BlockSpec((blk, d_feat), lambda i: (i, 0)),
          pl.BlockSpec((d_feat, dim_h), lambda i: (0, 0)),
          pl.BlockSpec((blk, 1), lambda i: (i, 0)),
          pl.BlockSpec((blk, 1), lambda i: (i, 0)),
      ],
      out_specs=pl.BlockSpec((blk, dim_h), lambda i: (i, 0)),
      out_shape=jax.ShapeDtypeStruct((n, dim_h), jnp.float32),
  )(x, W1, dega, degb)

  zrows = jnp.zeros((rpt, dim_h), jnp.float32)
  agg_parts = _make_agg_kernel(nchunk, nrows, rpt, dim_h)(
      hp, src2d, dst2d, zrows)
  sa = agg_parts[0, :n, :]
  sb = agg_parts[1, :n, :]

  bf = batch.astype(jnp.float32).reshape(n, 1)
  out = pl.pallas_call(
      _head_body,
      grid=(nb,),
      in_specs=[
          pl.BlockSpec((blk, dim_h), lambda i: (i, 0)),
          pl.BlockSpec((blk, dim_h), lambda i: (i, 0)),
          pl.BlockSpec((blk, dim_h), lambda i: (i, 0)),
          pl.BlockSpec((blk, 1), lambda i: (i, 0)),
          pl.BlockSpec((blk, 1), lambda i: (i, 0)),
          pl.BlockSpec((blk, 1), lambda i: (i, 0)),
          pl.BlockSpec((1, dim_h), lambda i: (0, 0)),
          pl.BlockSpec((dim_h, n_classes), lambda i: (0, 0)),
          pl.BlockSpec((1, n_classes), lambda i: (0, 0)),
      ],
      out_specs=pl.BlockSpec((n_graphs, n_classes), lambda i: (0, 0)),
      out_shape=jax.ShapeDtypeStruct((n_graphs, n_classes), jnp.float32),
      scratch_shapes=[
          pltpu.VMEM((n_graphs, dim_h), jnp.float32),
          pltpu.VMEM((1, n_graphs), jnp.float32),
      ],
  )(sa, sb, hp, dega, degb, bf, b1.reshape(1, dim_h), W2,
    b2.reshape(1, n_classes))
  return out


# speedup vs baseline: 7.6716x; 1.0840x over previous
import functools

import jax
import jax.numpy as jnp
from jax import lax
from jax.experimental import pallas as pl
from jax.experimental.pallas import tpu as pltpu
from jax.experimental.pallas import tpu_sc as plsc

NC = 2
NS = 16
NW = NC * NS
CHUNK = 128


def _mesh():
  return plsc.VectorSubcoreMesh(core_axis_name="c", subcore_axis_name="s")


def _make_deg_kernel(nchunk, nrows, rpt):
  @functools.partial(
      pl.kernel,
      out_type=jax.ShapeDtypeStruct((NC, nrows), jnp.float32),
      mesh=_mesh(),
      compiler_params=pltpu.CompilerParams(use_tc_tiling_on_sc=False),
      scratch_types=[
          pltpu.VMEM((nchunk, CHUNK), jnp.int32),
          pltpu.VMEM((CHUNK,), jnp.float32),
          pltpu.VMEM_SHARED((nrows,), jnp.float32),
      ],
  )
  def deg_kernel(dst_hbm, zrow_hbm, ones_hbm, out_hbm, idx_v, ones_v, deg_sh):
    c = lax.axis_index("c")
    s = lax.axis_index("s")
    w = c * NS + s
    pltpu.sync_copy(dst_hbm.at[pl.ds(w * nchunk, nchunk)], idx_v)
    pltpu.sync_copy(ones_hbm, ones_v)
    pltpu.sync_copy(zrow_hbm, deg_sh.at[pl.ds(s * rpt, rpt)])
    plsc.subcore_barrier()

    def chunk_body(j, carry):
      pltpu.sync_copy(ones_v, deg_sh.at[idx_v.at[j]], add=True)
      return carry

    lax.fori_loop(0, nchunk, chunk_body, 0)
    plsc.subcore_barrier()
    pltpu.sync_copy(deg_sh.at[pl.ds(s * rpt, rpt)],
                    out_hbm.at[c, pl.ds(s * rpt, rpt)])

  return deg_kernel


def _make_agg_kernel(nchunk, nrows, rpt, dim_h):
  nhalf = nchunk // 2

  @functools.partial(
      pl.kernel,
      out_type=jax.ShapeDtypeStruct((NC, nrows, dim_h), jnp.float32),
      mesh=_mesh(),
      scratch_types=[
          pltpu.VMEM((nhalf, CHUNK), jnp.int32),
          pltpu.VMEM((nhalf, CHUNK), jnp.int32),
          pltpu.VMEM((CHUNK, dim_h), jnp.float32),
          pltpu.VMEM((CHUNK, dim_h), jnp.float32),
          pltpu.VMEM_SHARED((nrows, dim_h), jnp.float32),
          pltpu.SemaphoreType.DMA,
          pltpu.SemaphoreType.DMA,
      ],
  )
  def agg_kernel(hp_hbm, src_hbm, dst_hbm, zrows_hbm, out_hbm,
                 sidx_v, didx_v, buf0, buf1, agg_sh, gsem0, gsem1):
    c = lax.axis_index("c")
    s = lax.axis_index("s")
    w = c * NS + s
    pltpu.sync_copy(zrows_hbm, agg_sh.at[pl.ds(s * rpt, rpt)])
    plsc.subcore_barrier()

    bufs = (buf0, buf1)
    gsems = (gsem0, gsem1)

    def wait_gather(b, j):
      pltpu.make_async_copy(hp_hbm.at[sidx_v.at[j]], bufs[b], gsems[b]).wait()

    for h in range(2):
      base = w * nchunk + h * nhalf
      pltpu.sync_copy(src_hbm.at[pl.ds(base, nhalf)], sidx_v)
      pltpu.sync_copy(dst_hbm.at[pl.ds(base, nhalf)], didx_v)
      pltpu.async_copy(hp_hbm.at[sidx_v.at[0]], buf0, gsem0)

      def pair_body(i, carry):
        for b in range(2):
          j = 2 * i + b
          nb = 1 - b
          wait_gather(b, j)

          @pl.when(j + 1 < nhalf)
          def _():
            pltpu.async_copy(hp_hbm.at[sidx_v.at[j + 1]], bufs[nb], gsems[nb])

          pltpu.sync_copy(bufs[b], agg_sh.at[didx_v.at[j]], add=True)
        return carry

      lax.fori_loop(0, nhalf // 2, pair_body, 0)

    plsc.subcore_barrier()
    pltpu.sync_copy(agg_sh.at[pl.ds(s * rpt, rpt)],
                    out_hbm.at[c, pl.ds(s * rpt, rpt)])

  return agg_kernel


def _mm_body(x_ref, w1_ref, o_ref):
  o_ref[...] = jnp.dot(x_ref[...], w1_ref[...],
                       preferred_element_type=jnp.float32)


def _scale_body(h_ref, dega_ref, degb_ref, o_ref):
  deg = dega_ref[...] + degb_ref[...] + 1.0
  o_ref[...] = h_ref[...] * lax.rsqrt(deg)


def _head_body(sa_ref, sb_ref, hp_ref, dega_ref, degb_ref, bf_ref, b1_ref,
               w2_ref, b2_ref, o_ref, sums, counts):
  j = pl.program_id(0)

  @pl.when(j == 0)
  def _init():
    sums[...] = jnp.zeros_like(sums)
    counts[...] = jnp.zeros_like(counts)

  d = lax.rsqrt(dega_ref[...] + degb_ref[...] + 1.0)
  h2 = d * (sa_ref[...] + sb_ref[...] + hp_ref[...]) + b1_ref[...]
  h2 = jnp.maximum(h2, 0.0)
  gids = lax.broadcasted_iota(jnp.int32, (1, sums.shape[0]), 1
                              ).astype(jnp.float32)
  onehot = (bf_ref[...] == gids).astype(jnp.float32)
  sums[...] += jnp.dot(onehot.T, h2, preferred_element_type=jnp.float32)
  counts[...] += jnp.sum(onehot, axis=0, keepdims=True)

  @pl.when(j == pl.num_programs(0) - 1)
  def _finish():
    hg = sums[...] / jnp.maximum(counts[...], 1.0).T
    logits = jnp.dot(hg, w2_ref[...],
                     preferred_element_type=jnp.float32) + b2_ref[...]
    m = jnp.max(logits, axis=1, keepdims=True)
    lse = jnp.log(jnp.sum(jnp.exp(logits - m), axis=1, keepdims=True)) + m
    o_ref[...] = logits - lse


def kernel(x, edge_index, batch, W1, b1, W2, b2):
  n, d_feat = x.shape
  dim_h = W1.shape[1]
  n_classes = W2.shape[1]
  e = edge_index.shape[1]
  n_graphs = 128

  nchunk = -(-e // (NW * CHUNK))
  nchunk = (nchunk + 7) // 8 * 8
  e_pad = NW * CHUNK * nchunk
  src = edge_index[0].astype(jnp.int32)
  dst = edge_index[1].astype(jnp.int32)
  pad = e_pad - e
  src2d = jnp.concatenate([src, jnp.zeros((pad,), jnp.int32)]
                          ).reshape(NW * nchunk, CHUNK)
  dst2d = jnp.concatenate([dst, jnp.full((pad,), n, jnp.int32)]
                          ).reshape(NW * nchunk, CHUNK)

  rpt = ((-(-(n + 1) // NS)) + 7) // 8 * 8
  nrows = rpt * NS

  zrow = jnp.zeros((rpt,), jnp.float32)
  ones = jnp.ones((CHUNK,), jnp.float32)
  deg_parts = _make_deg_kernel(nchunk, nrows, rpt)(dst2d, zrow, ones)
  dega = deg_parts[0, :n].reshape(n, 1)
  degb = deg_parts[1, :n].reshape(n, 1)

  nb = 10
  blk = n // nb
  h1 = pl.pallas_call(
      _mm_body,
      grid=(nb,),
      in_specs=[
          pl.BlockSpec((blk, d_feat), lambda i: (i, 0)),
          pl.BlockSpec((d_feat, dim_h), lambda i: (0, 0)),
      ],
      out_specs=pl.BlockSpec((blk, dim_h), lambda i: (i, 0)),
      out_shape=jax.ShapeDtypeStruct((n, dim_h), jnp.float32),
  )(x, W1)
  hp = pl.pallas_call(
      _scale_body,
      grid=(nb,),
      in_specs=[
          pl.BlockSpec((blk, dim_h), lambda i: (i, 0)),
          pl.BlockSpec((blk, 1), lambda i: (i, 0)),
          pl.BlockSpec((blk, 1), lambda i: (i, 0)),
      ],
      out_specs=pl.BlockSpec((blk, dim_h), lambda i: (i, 0)),
      out_shape=jax.ShapeDtypeStruct((n, dim_h), jnp.float32),
  )(h1, dega, degb)

  zrows = jnp.zeros((rpt, dim_h), jnp.float32)
  agg_parts = _make_agg_kernel(nchunk, nrows, rpt, dim_h)(
      hp, src2d, dst2d, zrows)
  sa = agg_parts[0, :n, :]
  sb = agg_parts[1, :n, :]

  bf = batch.astype(jnp.float32).reshape(n, 1)
  out = pl.pallas_call(
      _head_body,
      grid=(nb,),
      in_specs=[
          pl.BlockSpec((blk, dim_h), lambda i: (i, 0)),
          pl.BlockSpec((blk, dim_h), lambda i: (i, 0)),
          pl.BlockSpec((blk, dim_h), lambda i: (i, 0)),
          pl.BlockSpec((blk, 1), lambda i: (i, 0)),
          pl.BlockSpec((blk, 1), lambda i: (i, 0)),
          pl.BlockSpec((blk, 1), lambda i: (i, 0)),
          pl.BlockSpec((1, dim_h), lambda i: (0, 0)),
          pl.BlockSpec((dim_h, n_classes), lambda i: (0, 0)),
          pl.BlockSpec((1, n_classes), lambda i: (0, 0)),
      ],
      out_specs=pl.BlockSpec((n_graphs, n_classes), lambda i: (0, 0)),
      out_shape=jax.ShapeDtypeStruct((n_graphs, n_classes), jnp.float32),
      scratch_shapes=[
          pltpu.VMEM((n_graphs, dim_h), jnp.float32),
          pltpu.VMEM((1, n_graphs), jnp.float32),
      ],
  )(sa, sb, hp, dega, degb, bf, b1.reshape(1, dim_h), W2,
    b2.reshape(1, n_classes))
  return out
